# baseline (speedup 1.0000x reference)
import functools

import jax
import jax.numpy as jnp
from jax import lax
from jax.experimental import pallas as pl
from jax.experimental.pallas import tpu as pltpu
from jax.experimental.pallas import tpu_sc as plsc

N = 10000
E = 160000
F_IN, H, C = 256, 512, 128

NT = 16
EPT = 10240
E_PAD = NT * EPT

BW = 64
ND = 4
NBT = EPT // BW
NW = 4
WNB = NBT // NW

DBW = 128
DNB = EPT // DBW // 2

NB2 = NBT // 2
NW2 = NB2 // WNB

ACC_R = 10240
RPT_Z = 640
RPT_W = 624
DEG_R = 10240


def _deg_body(dst_hbm, out0_hbm, out1_hbm, dstv, ones_row, init_v, deg_sh):
    c = lax.axis_index("c")
    s = lax.axis_index("s")
    one16 = jnp.full((16,), 1.0, jnp.float32)
    iv = jnp.zeros((16,), jnp.float32) + jnp.where(
        c == 0, jnp.float32(1.0), jnp.float32(0.0))

    def fill_row(i, carry):
        ones_row[pl.ds(i * 16, 16)] = one16
        return carry

    lax.fori_loop(0, DBW // 16, fill_row, 0)

    def fill_init(i, carry):
        init_v[pl.ds(i * 16, 16)] = iv
        return carry

    lax.fori_loop(0, 640 // 16, fill_init, 0)

    pltpu.sync_copy(init_v, deg_sh.at[pl.ds(s * 640, 640)])
    pltpu.sync_copy(dst_hbm.at[c, s], dstv)
    plsc.subcore_barrier()

    def body(j, carry):
        pltpu.sync_copy(ones_row, deg_sh.at[dstv.at[j]], add=True)
        return carry

    lax.fori_loop(0, DNB, body, 0)
    plsc.subcore_barrier()

    @pl.when(c == 0)
    def _():
        pltpu.sync_copy(deg_sh.at[pl.ds(s * 640, 640)],
                        out0_hbm.at[pl.ds(s * 640, 640)])

    @pl.when(c == 1)
    def _():
        pltpu.sync_copy(deg_sh.at[pl.ds(s * 640, 640)],
                        out1_hbm.at[pl.ds(s * 640, 640)])


@functools.cache
def _deg_kernel_build():
    return pl.kernel(
        _deg_body,
        out_type=[jax.ShapeDtypeStruct((DEG_R,), jnp.float32),
                  jax.ShapeDtypeStruct((DEG_R,), jnp.float32)],
        mesh=plsc.VectorSubcoreMesh(core_axis_name="c", subcore_axis_name="s"),
        scratch_types=[
            pltpu.VMEM((DNB, DBW), jnp.int32),
            pltpu.VMEM((DBW,), jnp.float32),
            pltpu.VMEM((640,), jnp.float32),
            pltpu.VMEM_SHARED((DEG_R,), jnp.float32),
        ],
    )


def _deg_kernel(dst_t):
    return _deg_kernel_build()(dst_t)


def _edge_pipe(h_hbm, srcv, dstv, acc_sh, bufs, gsems, ssems, nb):
    def g_start(j, b):
        pltpu.async_copy(h_hbm.at[srcv.at[j]], bufs[b], gsems[b])

    def g_wait(j, b):
        pltpu.make_async_copy(h_hbm.at[srcv.at[j]], bufs[b], gsems[b]).wait()

    def s_start(j, b):
        pltpu.async_copy(bufs[b], acc_sh.at[dstv.at[j]], ssems[b], add=True)

    def s_wait(j, b):
        pltpu.make_async_copy(bufs[b], acc_sh.at[dstv.at[j]], ssems[b]).wait()

    for b in range(ND):
        g_start(b, b)
    g_wait(0, 0)
    s_start(0, 0)
    for j in range(1, ND):
        g_wait(j, j)
        s_wait(j - 1, j - 1)
        g_start(j + ND - 1, j - 1)
        s_start(j, j)

    def step(t, carry):
        for b in range(ND):
            j = ND * t + b
            g_wait(j, b)
            s_wait(j - 1, (b - 1) % ND)
            g_start(j + ND - 1, (b - 1) % ND)
            s_start(j, b)
        return carry

    lax.fori_loop(1, nb // ND - 1, step, 0)
    j0 = nb - ND
    g_wait(j0, 0)
    s_wait(j0 - 1, ND - 1)
    g_start(nb - 1, ND - 1)
    s_start(j0, 0)
    for b in range(1, ND):
        g_wait(j0 + b, b)
        s_start(j0 + b, b)
    for b in range(ND):
        s_wait(j0 + b, b)


def _zero_acc_slice(buf0, acc_sh, base):
    z16 = jnp.zeros((16,), jnp.float32)

    def zrow(i, carry):
        for k in range(128 // 16):
            buf0[i, pl.ds(k * 16, 16)] = z16
        return carry

    lax.fori_loop(0, BW, zrow, 0)
    for m in range(RPT_Z // BW):
        pltpu.sync_copy(buf0, acc_sh.at[pl.ds(base + m * BW, BW)])


def _writeback(acc_sh, out_view, s):
    wb = s * RPT_W

    @pl.when(s < NT - 1)
    def _():
        pltpu.sync_copy(acc_sh.at[pl.ds(wb, RPT_W)],
                        out_view.at[pl.ds(wb, RPT_W)])

    @pl.when(s == NT - 1)
    def _():
        pltpu.sync_copy(acc_sh.at[pl.ds((NT - 1) * RPT_W, N - (NT - 1) * RPT_W)],
                        out_view.at[pl.ds((NT - 1) * RPT_W, N - (NT - 1) * RPT_W)])


def _agg1_body(h_hbm, srcoff_hbm, dst_hbm, out_hbm, srcv, dstv,
               buf0, buf1, buf2, buf3, gs0, gs1, gs2, gs3, ss0, ss1, ss2, ss3,
               acc_sh):
    bufs = [buf0, buf1, buf2, buf3]
    gsems = [gs0, gs1, gs2, gs3]
    ssems = [ss0, ss1, ss2, ss3]
    c = lax.axis_index("c")
    s = lax.axis_index("s")

    for chunk in range(2):
        g = c * 2 + chunk
        _zero_acc_slice(buf0, acc_sh, s * RPT_Z)
        plsc.subcore_barrier()
        for wnd in range(NW):
            pltpu.sync_copy(srcoff_hbm.at[g, s, wnd], srcv)
            pltpu.sync_copy(dst_hbm.at[s, wnd], dstv)
            _edge_pipe(h_hbm, srcv, dstv, acc_sh, bufs, gsems, ssems, WNB)
        plsc.subcore_barrier()
        _writeback(acc_sh, out_hbm.at[g], s)
        plsc.subcore_barrier()


@functools.cache
def _agg1_build():
    return pl.kernel(
        _agg1_body,
        out_type=jax.ShapeDtypeStruct((4, N, 128), jnp.float32),
        mesh=plsc.VectorSubcoreMesh(core_axis_name="c", subcore_axis_name="s"),
        scratch_types=(
            [pltpu.VMEM((WNB, BW), jnp.int32),
             pltpu.VMEM((WNB, BW), jnp.int32)]
            + [pltpu.VMEM((BW, 128), jnp.float32)] * ND
            + [pltpu.SemaphoreType.DMA] * (2 * ND)
            + [pltpu.VMEM_SHARED((ACC_R, 128), jnp.float32)]
        ),
    )


def _agg1(hflat, src_c, dst_t):
    return _agg1_build()(hflat, src_c, dst_t)


def _agg2_body(h_hbm, src_hbm, dst_hbm, out_hbm, srcv, dstv,
               buf0, buf1, buf2, buf3, gs0, gs1, gs2, gs3, ss0, ss1, ss2, ss3,
               acc_sh):
    bufs = [buf0, buf1, buf2, buf3]
    gsems = [gs0, gs1, gs2, gs3]
    ssems = [ss0, ss1, ss2, ss3]
    c = lax.axis_index("c")
    s = lax.axis_index("s")
    _zero_acc_slice(buf0, acc_sh, s * RPT_Z)
    plsc.subcore_barrier()
    for wnd in range(NW2):
        pltpu.sync_copy(src_hbm.at[c, s, wnd], srcv)
        pltpu.sync_copy(dst_hbm.at[c, s, wnd], dstv)
        _edge_pipe(h_hbm, srcv, dstv, acc_sh, bufs, gsems, ssems, WNB)
    plsc.subcore_barrier()
    _writeback(acc_sh, out_hbm.at[c], s)


@functools.cache
def _agg2_build():
    return pl.kernel(
        _agg2_body,
        out_type=jax.ShapeDtypeStruct((2, N, C), jnp.float32),
        mesh=plsc.VectorSubcoreMesh(core_axis_name="c", subcore_axis_name="s"),
        scratch_types=(
            [pltpu.VMEM((WNB, BW), jnp.int32),
             pltpu.VMEM((WNB, BW), jnp.int32)]
            + [pltpu.VMEM((BW, C), jnp.float32)] * ND
            + [pltpu.SemaphoreType.DMA] * (2 * ND)
            + [pltpu.VMEM_SHARED((ACC_R, C), jnp.float32)]
        ),
    )


def _agg2(h2p, src_t2, dst_t2):
    return _agg2_build()(h2p, src_t2, dst_t2)


_RB = 2000


def _mm1_body(x_ref, w_ref, p0_ref, p1_ref, h_ref, dinv_ref):
    dinv = lax.rsqrt(p0_ref[...] + p1_ref[...])
    h = jnp.dot(x_ref[...].astype(jnp.bfloat16),
                w_ref[...].astype(jnp.bfloat16),
                preferred_element_type=jnp.float32)
    h_ref[0] = h * dinv
    dinv_ref[...] = dinv


def _mm1(x, w1, p0, p1):
    return pl.pallas_call(
        _mm1_body,
        grid=(N // _RB, H // 128),
        in_specs=[
            pl.BlockSpec((_RB, F_IN), lambda i, j: (i, 0)),
            pl.BlockSpec((F_IN, 128), lambda i, j: (0, j)),
            pl.BlockSpec((_RB, 1), lambda i, j: (i, 0)),
            pl.BlockSpec((_RB, 1), lambda i, j: (i, 0)),
        ],
        out_specs=[
            pl.BlockSpec((1, _RB, 128), lambda i, j: (j, i, 0)),
            pl.BlockSpec((_RB, 1), lambda i, j: (i, 0)),
        ],
        out_shape=[
            jax.ShapeDtypeStruct((H // 128, N, 128), jnp.float32),
            jax.ShapeDtypeStruct((N, 1), jnp.float32),
        ],
    )(x, w1, p0, p1)


def _mm2_body(s1_ref, h1_ref, dinv_ref, b1_ref, w2_ref, out_ref, acc_ref):
    k = pl.program_id(1)
    dinv = dinv_ref[...]
    u = jnp.maximum(dinv * (s1_ref[0] + h1_ref[0]) + b1_ref[...], 0.0)
    part = jnp.dot(u.astype(jnp.bfloat16),
                   w2_ref[...].astype(jnp.bfloat16),
                   preferred_element_type=jnp.float32)

    @pl.when(k == 0)
    def _():
        acc_ref[...] = part

    @pl.when(k > 0)
    def _():
        acc_ref[...] += part

    @pl.when(k == H // 128 - 1)
    def _():
        out_ref[...] = dinv * acc_ref[...]


def _mm2(s1, h1p, dinv, b1, w2):
    return pl.pallas_call(
        _mm2_body,
        grid=(N // _RB, H // 128),
        in_specs=[
            pl.BlockSpec((1, _RB, 128), lambda i, k: (k, i, 0)),
            pl.BlockSpec((1, _RB, 128), lambda i, k: (k, i, 0)),
            pl.BlockSpec((_RB, 1), lambda i, k: (i, 0)),
            pl.BlockSpec((1, 128), lambda i, k: (0, k)),
            pl.BlockSpec((128, C), lambda i, k: (k, 0)),
        ],
        out_specs=pl.BlockSpec((_RB, C), lambda i, k: (i, 0)),
        out_shape=jax.ShapeDtypeStruct((N, C), jnp.float32),
        scratch_shapes=[pltpu.VMEM((_RB, C), jnp.float32)],
    )(s1, h1p, dinv, b1, w2)


def _fin_body(s2_ref, h2_ref, dinv_ref, b2_ref, out_ref):
    dinv = dinv_ref[...]
    z = dinv * (s2_ref[0] + s2_ref[1] + h2_ref[...]) + b2_ref[...]
    m = jnp.max(z, axis=1, keepdims=True)
    lse = m + jnp.log(jnp.sum(jnp.exp(z - m), axis=1, keepdims=True))
    out_ref[...] = z - lse


def _fin(s2, h2p, dinv, b2):
    return pl.pallas_call(
        _fin_body,
        grid=(N // _RB,),
        in_specs=[
            pl.BlockSpec((2, _RB, C), lambda i: (0, i, 0)),
            pl.BlockSpec((_RB, C), lambda i: (i, 0)),
            pl.BlockSpec((_RB, 1), lambda i: (i, 0)),
            pl.BlockSpec((1, C), lambda i: (0, 0)),
        ],
        out_specs=pl.BlockSpec((_RB, C), lambda i: (i, 0)),
        out_shape=jax.ShapeDtypeStruct((N, C), jnp.float32),
    )(s2, h2p, dinv, b2)


def kernel(x, edge_index, W1, b1, W2, b2):
    src = edge_index[0].astype(jnp.int32)
    dst = edge_index[1].astype(jnp.int32)
    npad = E_PAD - E
    ar = jnp.arange(npad, dtype=jnp.int32)
    src_p = jnp.concatenate([src, (ar * 37) % N])
    dst_p = jnp.concatenate([dst, N + (ar % 16)])
    dst_t = dst_p.reshape(2, NT, DNB, DBW)
    dst_t1 = dst_p.reshape(NT, NW, WNB, BW)
    src4 = (src_p[None] + (jnp.arange(4, dtype=jnp.int32) * N)[:, None]
            ).reshape(4, NT, NW, WNB, BW)
    src_t2 = src_p.reshape(2, NT, NW2, WNB, BW)
    dst_t2 = dst_p.reshape(2, NT, NW2, WNB, BW)

    p0, p1 = _deg_kernel(dst_t)
    h1p, dinv = _mm1(x, W1, p0.reshape(DEG_R, 1), p1.reshape(DEG_R, 1))
    s1 = _agg1(h1p.reshape(4 * N, 128), src4, dst_t1)
    h2p = _mm2(s1, h1p, dinv, b1.reshape(1, H), W2)
    s2 = _agg2(h2p, src_t2, dst_t2)
    return _fin(s2, h2p, dinv, b2.reshape(1, C))


# speedup vs baseline: 1.0126x; 1.0126x over previous
import functools

import jax
import jax.numpy as jnp
from jax import lax
from jax.experimental import pallas as pl
from jax.experimental.pallas import tpu as pltpu
from jax.experimental.pallas import tpu_sc as plsc

N = 10000
E = 160000
F_IN, H, C = 256, 512, 128

NT = 16
EPT = 10240
E_PAD = NT * EPT

BW = 64
ND = 4
NBT = EPT // BW
NW = 4
WNB = NBT // NW

DBW = 128
DNB = EPT // DBW // 2

NB2 = NBT // 2
NW2 = NB2 // WNB

ACC_R = 10240
RPT_Z = 640
RPT_W = 624
DEG_R = 10240


def _deg_body(dst_hbm, out0_hbm, out1_hbm, dstv, ones_row, init_v, deg_sh):
    c = lax.axis_index("c")
    s = lax.axis_index("s")
    one16 = jnp.full((16,), 1.0, jnp.float32)
    iv = jnp.zeros((16,), jnp.float32) + jnp.where(
        c == 0, jnp.float32(1.0), jnp.float32(0.0))

    def fill_row(i, carry):
        ones_row[pl.ds(i * 16, 16)] = one16
        return carry

    lax.fori_loop(0, DBW // 16, fill_row, 0)

    def fill_init(i, carry):
        init_v[pl.ds(i * 16, 16)] = iv
        return carry

    lax.fori_loop(0, 640 // 16, fill_init, 0)

    pltpu.sync_copy(init_v, deg_sh.at[pl.ds(s * 640, 640)])
    pltpu.sync_copy(dst_hbm.at[c, s], dstv)
    plsc.subcore_barrier()

    def body(j, carry):
        pltpu.sync_copy(ones_row, deg_sh.at[dstv.at[j]], add=True)
        return carry

    lax.fori_loop(0, DNB, body, 0)
    plsc.subcore_barrier()

    @pl.when(c == 0)
    def _():
        pltpu.sync_copy(deg_sh.at[pl.ds(s * 640, 640)],
                        out0_hbm.at[pl.ds(s * 640, 640)])

    @pl.when(c == 1)
    def _():
        pltpu.sync_copy(deg_sh.at[pl.ds(s * 640, 640)],
                        out1_hbm.at[pl.ds(s * 640, 640)])


@functools.cache
def _deg_kernel_build():
    return pl.kernel(
        _deg_body,
        out_type=[jax.ShapeDtypeStruct((DEG_R,), jnp.float32),
                  jax.ShapeDtypeStruct((DEG_R,), jnp.float32)],
        mesh=plsc.VectorSubcoreMesh(core_axis_name="c", subcore_axis_name="s"),
        scratch_types=[
            pltpu.VMEM((DNB, DBW), jnp.int32),
            pltpu.VMEM((DBW,), jnp.float32),
            pltpu.VMEM((640,), jnp.float32),
            pltpu.VMEM_SHARED((DEG_R,), jnp.float32),
        ],
    )


def _deg_kernel(dst_t):
    return _deg_kernel_build()(dst_t)


def _edge_pipe(h_hbm, srcv, dstv, acc_sh, bufs, gsems, ssems, nb):
    def g_start(j, b):
        pltpu.async_copy(h_hbm.at[srcv.at[j]], bufs[b], gsems[b])

    def g_wait(j, b):
        pltpu.make_async_copy(h_hbm.at[srcv.at[j]], bufs[b], gsems[b]).wait()

    def s_start(j, b):
        pltpu.async_copy(bufs[b], acc_sh.at[dstv.at[j]], ssems[b], add=True)

    def s_wait(j, b):
        pltpu.make_async_copy(bufs[b], acc_sh.at[dstv.at[j]], ssems[b]).wait()

    for b in range(ND):
        g_start(b, b)
    g_wait(0, 0)
    s_start(0, 0)
    for j in range(1, ND):
        g_wait(j, j)
        s_wait(j - 1, j - 1)
        g_start(j + ND - 1, j - 1)
        s_start(j, j)

    def step(t, carry):
        for b in range(ND):
            j = ND * t + b
            g_wait(j, b)
            s_wait(j - 1, (b - 1) % ND)
            g_start(j + ND - 1, (b - 1) % ND)
            s_start(j, b)
        return carry

    lax.fori_loop(1, nb // ND - 1, step, 0)
    j0 = nb - ND
    g_wait(j0, 0)
    s_wait(j0 - 1, ND - 1)
    g_start(nb - 1, ND - 1)
    s_start(j0, 0)
    for b in range(1, ND):
        g_wait(j0 + b, b)
        s_start(j0 + b, b)
    for b in range(ND):
        s_wait(j0 + b, b)


def _zero_acc_slice(buf0, acc_sh, base):
    z16 = jnp.zeros((16,), jnp.float32)

    def zrow(i, carry):
        for k in range(128 // 16):
            buf0[i, pl.ds(k * 16, 16)] = z16
        return carry

    lax.fori_loop(0, BW, zrow, 0)
    for m in range(RPT_Z // BW):
        pltpu.sync_copy(buf0, acc_sh.at[pl.ds(base + m * BW, BW)])


def _writeback(acc_sh, out_view, s):
    wb = s * RPT_W

    @pl.when(s < NT - 1)
    def _():
        pltpu.sync_copy(acc_sh.at[pl.ds(wb, RPT_W)],
                        out_view.at[pl.ds(wb, RPT_W)])

    @pl.when(s == NT - 1)
    def _():
        pltpu.sync_copy(acc_sh.at[pl.ds((NT - 1) * RPT_W, N - (NT - 1) * RPT_W)],
                        out_view.at[pl.ds((NT - 1) * RPT_W, N - (NT - 1) * RPT_W)])


@functools.cache
def _agg1_build(ph):
    def body(h_hbm, srcoff_hbm, dst_hbm, out_hbm, srcv, dstv,
             buf0, buf1, buf2, buf3, gs0, gs1, gs2, gs3, ss0, ss1, ss2, ss3,
             acc_sh):
        bufs = [buf0, buf1, buf2, buf3]
        gsems = [gs0, gs1, gs2, gs3]
        ssems = [ss0, ss1, ss2, ss3]
        c = lax.axis_index("c")
        s = lax.axis_index("s")
        g = c * 2 + ph
        _zero_acc_slice(buf0, acc_sh, s * RPT_Z)
        plsc.subcore_barrier()
        for wnd in range(NW):
            pltpu.sync_copy(srcoff_hbm.at[g, s, wnd], srcv)
            pltpu.sync_copy(dst_hbm.at[s, wnd], dstv)
            _edge_pipe(h_hbm, srcv, dstv, acc_sh, bufs, gsems, ssems, WNB)
        plsc.subcore_barrier()
        _writeback(acc_sh, out_hbm.at[c], s)

    return pl.kernel(
        body,
        out_type=jax.ShapeDtypeStruct((2, N, 128), jnp.float32),
        mesh=plsc.VectorSubcoreMesh(core_axis_name="c", subcore_axis_name="s"),
        scratch_types=(
            [pltpu.VMEM((WNB, BW), jnp.int32),
             pltpu.VMEM((WNB, BW), jnp.int32)]
            + [pltpu.VMEM((BW, 128), jnp.float32)] * ND
            + [pltpu.SemaphoreType.DMA] * (2 * ND)
            + [pltpu.VMEM_SHARED((ACC_R, 128), jnp.float32)]
        ),
    )


def _agg1(ph, hflat, src_c, dst_t):
    return _agg1_build(ph)(hflat, src_c, dst_t)


def _agg2_body(h_hbm, src_hbm, dst_hbm, out_hbm, srcv, dstv,
               buf0, buf1, buf2, buf3, gs0, gs1, gs2, gs3, ss0, ss1, ss2, ss3,
               acc_sh):
    bufs = [buf0, buf1, buf2, buf3]
    gsems = [gs0, gs1, gs2, gs3]
    ssems = [ss0, ss1, ss2, ss3]
    c = lax.axis_index("c")
    s = lax.axis_index("s")
    _zero_acc_slice(buf0, acc_sh, s * RPT_Z)
    plsc.subcore_barrier()
    for wnd in range(NW2):
        pltpu.sync_copy(src_hbm.at[c, s, wnd], srcv)
        pltpu.sync_copy(dst_hbm.at[c, s, wnd], dstv)
        _edge_pipe(h_hbm, srcv, dstv, acc_sh, bufs, gsems, ssems, WNB)
    plsc.subcore_barrier()
    _writeback(acc_sh, out_hbm.at[c], s)


@functools.cache
def _agg2_build():
    return pl.kernel(
        _agg2_body,
        out_type=jax.ShapeDtypeStruct((2, N, C), jnp.float32),
        mesh=plsc.VectorSubcoreMesh(core_axis_name="c", subcore_axis_name="s"),
        scratch_types=(
            [pltpu.VMEM((WNB, BW), jnp.int32),
             pltpu.VMEM((WNB, BW), jnp.int32)]
            + [pltpu.VMEM((BW, C), jnp.float32)] * ND
            + [pltpu.SemaphoreType.DMA] * (2 * ND)
            + [pltpu.VMEM_SHARED((ACC_R, C), jnp.float32)]
        ),
    )


def _agg2(h2p, src_t2, dst_t2):
    return _agg2_build()(h2p, src_t2, dst_t2)


_RB = 2000


def _mm1_body(x_ref, w_ref, p0_ref, p1_ref, h_ref, dinv_ref):
    dinv = lax.rsqrt(p0_ref[...] + p1_ref[...])
    h = jnp.dot(x_ref[...].astype(jnp.bfloat16),
                w_ref[...].astype(jnp.bfloat16),
                preferred_element_type=jnp.float32)
    h_ref[0] = h * dinv
    dinv_ref[...] = dinv


def _mm1(x, w1, p0, p1):
    return pl.pallas_call(
        _mm1_body,
        grid=(N // _RB, H // 128),
        in_specs=[
            pl.BlockSpec((_RB, F_IN), lambda i, j: (i, 0)),
            pl.BlockSpec((F_IN, 128), lambda i, j: (0, j)),
            pl.BlockSpec((_RB, 1), lambda i, j: (i, 0)),
            pl.BlockSpec((_RB, 1), lambda i, j: (i, 0)),
        ],
        out_specs=[
            pl.BlockSpec((1, _RB, 128), lambda i, j: (j, i, 0)),
            pl.BlockSpec((_RB, 1), lambda i, j: (i, 0)),
        ],
        out_shape=[
            jax.ShapeDtypeStruct((H // 128, N, 128), jnp.float32),
            jax.ShapeDtypeStruct((N, 1), jnp.float32),
        ],
    )(x, w1, p0, p1)


def _mm2_half(ph, final):
    def body(*refs):
        if final:
            s1_ref, h1_ref, dinv_ref, b1_ref, w2_ref, p_ref, out_ref, acc_ref = refs
        else:
            s1_ref, h1_ref, dinv_ref, b1_ref, w2_ref, out_ref, acc_ref = refs
        a = pl.program_id(1)
        dinv = dinv_ref[...]
        u = jnp.maximum(dinv * (s1_ref[0] + h1_ref[0]) + b1_ref[...], 0.0)
        part = jnp.dot(u.astype(jnp.bfloat16),
                       w2_ref[...].astype(jnp.bfloat16),
                       preferred_element_type=jnp.float32)

        @pl.when(a == 0)
        def _():
            acc_ref[...] = part

        @pl.when(a == 1)
        def _():
            if final:
                out_ref[...] = dinv * (acc_ref[...] + part + p_ref[...])
            else:
                out_ref[...] = acc_ref[...] + part

    in_specs = [
        pl.BlockSpec((1, _RB, 128), lambda i, a: (a, i, 0)),
        pl.BlockSpec((1, _RB, 128), lambda i, a: (2 * a + ph, i, 0)),
        pl.BlockSpec((_RB, 1), lambda i, a: (i, 0)),
        pl.BlockSpec((1, 128), lambda i, a: (0, 2 * a + ph)),
        pl.BlockSpec((128, C), lambda i, a: (2 * a + ph, 0)),
    ]
    if final:
        in_specs.append(pl.BlockSpec((_RB, C), lambda i, a: (i, 0)))
    return pl.pallas_call(
        body,
        grid=(N // _RB, 2),
        in_specs=in_specs,
        out_specs=pl.BlockSpec((_RB, C), lambda i, a: (i, 0)),
        out_shape=jax.ShapeDtypeStruct((N, C), jnp.float32),
        scratch_shapes=[pltpu.VMEM((_RB, C), jnp.float32)],
    )


def _mm2a(sA, h1p, dinv, b1, w2):
    return _mm2_half(0, False)(sA, h1p, dinv, b1, w2)


def _mm2b(sB, h1p, dinv, b1, w2, part):
    return _mm2_half(1, True)(sB, h1p, dinv, b1, w2, part)


def _fin_body(s2_ref, h2_ref, dinv_ref, b2_ref, out_ref):
    dinv = dinv_ref[...]
    z = dinv * (s2_ref[0] + s2_ref[1] + h2_ref[...]) + b2_ref[...]
    m = jnp.max(z, axis=1, keepdims=True)
    lse = m + jnp.log(jnp.sum(jnp.exp(z - m), axis=1, keepdims=True))
    out_ref[...] = z - lse


def _fin(s2, h2p, dinv, b2):
    return pl.pallas_call(
        _fin_body,
        grid=(N // _RB,),
        in_specs=[
            pl.BlockSpec((2, _RB, C), lambda i: (0, i, 0)),
            pl.BlockSpec((_RB, C), lambda i: (i, 0)),
            pl.BlockSpec((_RB, 1), lambda i: (i, 0)),
            pl.BlockSpec((1, C), lambda i: (0, 0)),
        ],
        out_specs=pl.BlockSpec((_RB, C), lambda i: (i, 0)),
        out_shape=jax.ShapeDtypeStruct((N, C), jnp.float32),
    )(s2, h2p, dinv, b2)


def kernel(x, edge_index, W1, b1, W2, b2):
    src = edge_index[0].astype(jnp.int32)
    dst = edge_index[1].astype(jnp.int32)
    npad = E_PAD - E
    ar = jnp.arange(npad, dtype=jnp.int32)
    src_p = jnp.concatenate([src, (ar * 37) % N])
    dst_p = jnp.concatenate([dst, N + (ar % 16)])
    dst_t = dst_p.reshape(2, NT, DNB, DBW)
    dst_t1 = dst_p.reshape(NT, NW, WNB, BW)
    src4 = (src_p[None] + (jnp.arange(4, dtype=jnp.int32) * N)[:, None]
            ).reshape(4, NT, NW, WNB, BW)
    src_t2 = src_p.reshape(2, NT, NW2, WNB, BW)
    dst_t2 = dst_p.reshape(2, NT, NW2, WNB, BW)

    p0, p1 = _deg_kernel(dst_t)
    h1p, dinv = _mm1(x, W1, p0.reshape(DEG_R, 1), p1.reshape(DEG_R, 1))
    h1flat = h1p.reshape(4 * N, 128)
    sA = _agg1(0, h1flat, src4, dst_t1)
    sB = _agg1(1, h1flat, src4, dst_t1)
    part = _mm2a(sA, h1p, dinv, b1.reshape(1, H), W2)
    h2p = _mm2b(sB, h1p, dinv, b1.reshape(1, H), W2, part)
    s2 = _agg2(h2p, src_t2, dst_t2)
    return _fin(s2, h2p, dinv, b2.reshape(1, C))


# speedup vs baseline: 1.0313x; 1.0185x over previous
import functools

import jax
import jax.numpy as jnp
from jax import lax
from jax.experimental import pallas as pl
from jax.experimental.pallas import tpu as pltpu
from jax.experimental.pallas import tpu_sc as plsc

N = 10000
E = 160000
F_IN, H, C = 256, 512, 128

NT = 16
EPT = 10240
E_PAD = NT * EPT

BW = 64
ND = 4
NBT = EPT // BW
NW = 4
WNB = NBT // NW

DBW = 128
DNB = EPT // DBW // 2

NB2 = NBT // 2
NW2 = NB2 // WNB

ACC_R = 10240
RPT_Z = 640
RPT_W = 624
DEG_R = 10240


def _deg_body(dst_hbm, out0_hbm, out1_hbm, dstv, ones_row, init_v, deg_sh):
    c = lax.axis_index("c")
    s = lax.axis_index("s")
    one16 = jnp.full((16,), 1.0, jnp.float32)
    iv = jnp.zeros((16,), jnp.float32) + jnp.where(
        c == 0, jnp.float32(1.0), jnp.float32(0.0))

    def fill_row(i, carry):
        ones_row[pl.ds(i * 16, 16)] = one16
        return carry

    lax.fori_loop(0, DBW // 16, fill_row, 0)

    def fill_init(i, carry):
        init_v[pl.ds(i * 16, 16)] = iv
        return carry

    lax.fori_loop(0, 640 // 16, fill_init, 0)

    pltpu.sync_copy(init_v, deg_sh.at[pl.ds(s * 640, 640)])
    pltpu.sync_copy(dst_hbm.at[c, s], dstv)
    plsc.subcore_barrier()

    def body(j, carry):
        pltpu.sync_copy(ones_row, deg_sh.at[dstv.at[j]], add=True)
        return carry

    lax.fori_loop(0, DNB, body, 0)
    plsc.subcore_barrier()

    @pl.when(c == 0)
    def _():
        pltpu.sync_copy(deg_sh.at[pl.ds(s * 640, 640)],
                        out0_hbm.at[pl.ds(s * 640, 640)])

    @pl.when(c == 1)
    def _():
        pltpu.sync_copy(deg_sh.at[pl.ds(s * 640, 640)],
                        out1_hbm.at[pl.ds(s * 640, 640)])


@functools.cache
def _deg_kernel_build():
    return pl.kernel(
        _deg_body,
        out_type=[jax.ShapeDtypeStruct((DEG_R,), jnp.float32),
                  jax.ShapeDtypeStruct((DEG_R,), jnp.float32)],
        mesh=plsc.VectorSubcoreMesh(core_axis_name="c", subcore_axis_name="s"),
        scratch_types=[
            pltpu.VMEM((DNB, DBW), jnp.int32),
            pltpu.VMEM((DBW,), jnp.float32),
            pltpu.VMEM((640,), jnp.float32),
            pltpu.VMEM_SHARED((DEG_R,), jnp.float32),
        ],
    )


def _deg_kernel(dst_t):
    return _deg_kernel_build()(dst_t)


def _edge_pipe(h_hbm, srcv, dstv, acc_sh, bufs, gsems, ssems, nb):
    def g_start(j, b):
        pltpu.async_copy(h_hbm.at[srcv.at[j]], bufs[b], gsems[b])

    def g_wait(j, b):
        pltpu.make_async_copy(h_hbm.at[srcv.at[j]], bufs[b], gsems[b]).wait()

    def s_start(j, b):
        pltpu.async_copy(bufs[b], acc_sh.at[dstv.at[j]], ssems[b], add=True)

    def s_wait(j, b):
        pltpu.make_async_copy(bufs[b], acc_sh.at[dstv.at[j]], ssems[b]).wait()

    for b in range(ND):
        g_start(b, b)
    g_wait(0, 0)
    s_start(0, 0)
    for j in range(1, ND):
        g_wait(j, j)
        s_wait(j - 1, j - 1)
        g_start(j + ND - 1, j - 1)
        s_start(j, j)

    def step(t, carry):
        for b in range(ND):
            j = ND * t + b
            g_wait(j, b)
            s_wait(j - 1, (b - 1) % ND)
            g_start(j + ND - 1, (b - 1) % ND)
            s_start(j, b)
        return carry

    lax.fori_loop(1, nb // ND - 1, step, 0)
    j0 = nb - ND
    g_wait(j0, 0)
    s_wait(j0 - 1, ND - 1)
    g_start(nb - 1, ND - 1)
    s_start(j0, 0)
    for b in range(1, ND):
        g_wait(j0 + b, b)
        s_start(j0 + b, b)
    for b in range(ND):
        s_wait(j0 + b, b)


def _zero_acc_slice(buf0, acc_sh, base):
    z16 = jnp.zeros((16,), jnp.float32)

    def zrow(i, carry):
        for k in range(128 // 16):
            buf0[i, pl.ds(k * 16, 16)] = z16
        return carry

    lax.fori_loop(0, BW, zrow, 0)
    for m in range(RPT_Z // BW):
        pltpu.sync_copy(buf0, acc_sh.at[pl.ds(base + m * BW, BW)])


def _writeback(acc_sh, out_view, s):
    wb = s * RPT_W

    @pl.when(s < NT - 1)
    def _():
        pltpu.sync_copy(acc_sh.at[pl.ds(wb, RPT_W)],
                        out_view.at[pl.ds(wb, RPT_W)])

    @pl.when(s == NT - 1)
    def _():
        pltpu.sync_copy(acc_sh.at[pl.ds((NT - 1) * RPT_W, N - (NT - 1) * RPT_W)],
                        out_view.at[pl.ds((NT - 1) * RPT_W, N - (NT - 1) * RPT_W)])


@functools.cache
def _agg1_build(ph):
    def body(h_hbm, sd_hbm, out_hbm, idxv,
             buf0, buf1, buf2, buf3, gs0, gs1, gs2, gs3, ss0, ss1, ss2, ss3,
             acc_sh):
        bufs = [buf0, buf1, buf2, buf3]
        gsems = [gs0, gs1, gs2, gs3]
        ssems = [ss0, ss1, ss2, ss3]
        c = lax.axis_index("c")
        s = lax.axis_index("s")
        g = c * 2 + ph
        _zero_acc_slice(buf0, acc_sh, s * RPT_Z)
        plsc.subcore_barrier()
        for wnd in range(NW):
            pltpu.sync_copy(sd_hbm.at[g, s, wnd], idxv)
            _edge_pipe(h_hbm, idxv.at[0], idxv.at[1], acc_sh,
                       bufs, gsems, ssems, WNB)
        plsc.subcore_barrier()
        _writeback(acc_sh, out_hbm.at[c], s)

    return pl.kernel(
        body,
        out_type=jax.ShapeDtypeStruct((2, N, 128), jnp.float32),
        mesh=plsc.VectorSubcoreMesh(core_axis_name="c", subcore_axis_name="s"),
        scratch_types=(
            [pltpu.VMEM((2, WNB, BW), jnp.int32)]
            + [pltpu.VMEM((BW, 128), jnp.float32)] * ND
            + [pltpu.SemaphoreType.DMA] * (2 * ND)
            + [pltpu.VMEM_SHARED((ACC_R, 128), jnp.float32)]
        ),
    )


def _agg1(ph, hflat, sd):
    return _agg1_build(ph)(hflat, sd)


def _agg2_body(h_hbm, sd_hbm, out_hbm, idxv,
               buf0, buf1, buf2, buf3, gs0, gs1, gs2, gs3, ss0, ss1, ss2, ss3,
               acc_sh):
    bufs = [buf0, buf1, buf2, buf3]
    gsems = [gs0, gs1, gs2, gs3]
    ssems = [ss0, ss1, ss2, ss3]
    c = lax.axis_index("c")
    s = lax.axis_index("s")
    _zero_acc_slice(buf0, acc_sh, s * RPT_Z)
    plsc.subcore_barrier()
    for wnd in range(NW2):
        pltpu.sync_copy(sd_hbm.at[c, s, wnd], idxv)
        _edge_pipe(h_hbm, idxv.at[0], idxv.at[1], acc_sh,
                   bufs, gsems, ssems, WNB)
    plsc.subcore_barrier()
    _writeback(acc_sh, out_hbm.at[c], s)


@functools.cache
def _agg2_build():
    return pl.kernel(
        _agg2_body,
        out_type=jax.ShapeDtypeStruct((2, N, C), jnp.float32),
        mesh=plsc.VectorSubcoreMesh(core_axis_name="c", subcore_axis_name="s"),
        scratch_types=(
            [pltpu.VMEM((2, WNB, BW), jnp.int32)]
            + [pltpu.VMEM((BW, C), jnp.float32)] * ND
            + [pltpu.SemaphoreType.DMA] * (2 * ND)
            + [pltpu.VMEM_SHARED((ACC_R, C), jnp.float32)]
        ),
    )


def _agg2(h2p, sd2):
    return _agg2_build()(h2p, sd2)


_RB = 2000


def _mm1_body(x_ref, w_ref, p0_ref, p1_ref, h_ref, dinv_ref):
    dinv = lax.rsqrt(p0_ref[...] + p1_ref[...])
    h = jnp.dot(x_ref[...].astype(jnp.bfloat16),
                w_ref[...].astype(jnp.bfloat16),
                preferred_element_type=jnp.float32)
    h_ref[0] = h * dinv
    dinv_ref[...] = dinv


def _mm1(x, w1, p0, p1):
    return pl.pallas_call(
        _mm1_body,
        grid=(N // _RB, H // 128),
        in_specs=[
            pl.BlockSpec((_RB, F_IN), lambda i, j: (i, 0)),
            pl.BlockSpec((F_IN, 128), lambda i, j: (0, j)),
            pl.BlockSpec((_RB, 1), lambda i, j: (i, 0)),
            pl.BlockSpec((_RB, 1), lambda i, j: (i, 0)),
        ],
        out_specs=[
            pl.BlockSpec((1, _RB, 128), lambda i, j: (j, i, 0)),
            pl.BlockSpec((_RB, 1), lambda i, j: (i, 0)),
        ],
        out_shape=[
            jax.ShapeDtypeStruct((H // 128, N, 128), jnp.float32),
            jax.ShapeDtypeStruct((N, 1), jnp.float32),
        ],
    )(x, w1, p0, p1)


def _mm2_half(ph, final):
    def body(*refs):
        if final:
            s1_ref, h1_ref, dinv_ref, b1_ref, w2_ref, p_ref, out_ref, acc_ref = refs
        else:
            s1_ref, h1_ref, dinv_ref, b1_ref, w2_ref, out_ref, acc_ref = refs
        a = pl.program_id(1)
        dinv = dinv_ref[...]
        u = jnp.maximum(dinv * (s1_ref[0] + h1_ref[0]) + b1_ref[...], 0.0)
        part = jnp.dot(u.astype(jnp.bfloat16),
                       w2_ref[...].astype(jnp.bfloat16),
                       preferred_element_type=jnp.float32)

        @pl.when(a == 0)
        def _():
            acc_ref[...] = part

        @pl.when(a == 1)
        def _():
            if final:
                out_ref[...] = dinv * (acc_ref[...] + part + p_ref[...])
            else:
                out_ref[...] = acc_ref[...] + part

    in_specs = [
        pl.BlockSpec((1, _RB, 128), lambda i, a: (a, i, 0)),
        pl.BlockSpec((1, _RB, 128), lambda i, a: (2 * a + ph, i, 0)),
        pl.BlockSpec((_RB, 1), lambda i, a: (i, 0)),
        pl.BlockSpec((1, 128), lambda i, a: (0, 2 * a + ph)),
        pl.BlockSpec((128, C), lambda i, a: (2 * a + ph, 0)),
    ]
    if final:
        in_specs.append(pl.BlockSpec((_RB, C), lambda i, a: (i, 0)))
    return pl.pallas_call(
        body,
        grid=(N // _RB, 2),
        in_specs=in_specs,
        out_specs=pl.BlockSpec((_RB, C), lambda i, a: (i, 0)),
        out_shape=jax.ShapeDtypeStruct((N, C), jnp.float32),
        scratch_shapes=[pltpu.VMEM((_RB, C), jnp.float32)],
    )


def _mm2a(sA, h1p, dinv, b1, w2):
    return _mm2_half(0, False)(sA, h1p, dinv, b1, w2)


def _mm2b(sB, h1p, dinv, b1, w2, part):
    return _mm2_half(1, True)(sB, h1p, dinv, b1, w2, part)


def _fin_body(s2_ref, h2_ref, dinv_ref, b2_ref, out_ref):
    dinv = dinv_ref[...]
    z = dinv * (s2_ref[0] + s2_ref[1] + h2_ref[...]) + b2_ref[...]
    m = jnp.max(z, axis=1, keepdims=True)
    lse = m + jnp.log(jnp.sum(jnp.exp(z - m), axis=1, keepdims=True))
    out_ref[...] = z - lse


def _fin(s2, h2p, dinv, b2):
    return pl.pallas_call(
        _fin_body,
        grid=(N // _RB,),
        in_specs=[
            pl.BlockSpec((2, _RB, C), lambda i: (0, i, 0)),
            pl.BlockSpec((_RB, C), lambda i: (i, 0)),
            pl.BlockSpec((_RB, 1), lambda i: (i, 0)),
            pl.BlockSpec((1, C), lambda i: (0, 0)),
        ],
        out_specs=pl.BlockSpec((_RB, C), lambda i: (i, 0)),
        out_shape=jax.ShapeDtypeStruct((N, C), jnp.float32),
    )(s2, h2p, dinv, b2)


def kernel(x, edge_index, W1, b1, W2, b2):
    src = edge_index[0].astype(jnp.int32)
    dst = edge_index[1].astype(jnp.int32)
    npad = E_PAD - E
    ar = jnp.arange(npad, dtype=jnp.int32)
    src_p = jnp.concatenate([src, (ar * 37) % N])
    dst_p = jnp.concatenate([dst, N + (ar % 16)])
    dst_t = dst_p.reshape(2, NT, DNB, DBW)
    dst_t1 = dst_p.reshape(NT, NW, WNB, BW)
    src4 = (src_p[None] + (jnp.arange(4, dtype=jnp.int32) * N)[:, None]
            ).reshape(4, NT, NW, WNB, BW)
    sd1 = jnp.stack(
        [src4, jnp.broadcast_to(dst_t1[None], src4.shape)], axis=3)
    sd2 = jnp.stack([src_p.reshape(2, NT, NW2, WNB, BW),
                     dst_p.reshape(2, NT, NW2, WNB, BW)], axis=3)

    p0, p1 = _deg_kernel(dst_t)
    h1p, dinv = _mm1(x, W1, p0.reshape(DEG_R, 1), p1.reshape(DEG_R, 1))
    h1flat = h1p.reshape(4 * N, 128)
    sA = _agg1(0, h1flat, sd1)
    sB = _agg1(1, h1flat, sd1)
    part = _mm2a(sA, h1p, dinv, b1.reshape(1, H), W2)
    h2p = _mm2b(sB, h1p, dinv, b1.reshape(1, H), W2, part)
    s2 = _agg2(h2p, sd2)
    return _fin(s2, h2p, dinv, b2.reshape(1, C))
